# baseline (speedup 1.0000x reference)
import functools

import jax
import jax.numpy as jnp
from jax import lax
from jax.experimental import pallas as pl
from jax.experimental.pallas import tpu as pltpu
from jax.experimental.pallas import tpu_sc as plsc

N = 100000
D = 128
NC = 2
NS = 16
NW = NC * NS
C = 128
BPW = N // NW
CPW = BPW // C
CTAIL = BPW - CPW * C
BPW_S = 3136
NST = NW * BPW_S

T1_ROWS = 120 * 16
T2_ROWS = 16 * 16 * 8
T3_ROWS = 13 * 16 * 4 * 2
NUM_GROUPS = 3


def _embed_body(idx_hbm, w0, w2p, w3, w6p, w4, w7, w8p, w1, w5, out_hbm,
                idx_v, cidx_v, acc0, acc1, spm1, spm2, spm3,
                sem_g0, sem_g1, sem_o0, sem_o1):
    accs = (acc0, acc1)
    gsems = (sem_g0, sem_g1)
    osems = (sem_o0, sem_o1)
    sid = lax.axis_index("s")
    wid = sid * NC + lax.axis_index("c")
    base = wid * BPW
    tables = (spm1, spm2, spm3)

    for f in range(9):
        pltpu.sync_copy(idx_hbm.at[pl.ds(f * NST + wid * BPW_S, BPW_S)],
                        idx_v.at[pl.ds(f * BPW_S, BPW_S)])

    R1, R2, R3 = T1_ROWS // NS, T2_ROWS // NS, T3_ROWS // NS
    iota = lax.iota(jnp.int32, 16)
    for j in range(8):
        o = j * 16
        r1 = sid * R1 + o + iota
        cidx_v[pl.ds(0 * C + o, 16)] = jnp.minimum(r1 >> 4, 118)
        cidx_v[pl.ds(1 * C + o, 16)] = r1 & 15
        r2 = sid * R2 + o + iota
        cidx_v[pl.ds(2 * C + o, 16)] = r2 >> 7
        cidx_v[pl.ds(3 * C + o, 16)] = (r2 >> 3) & 15
        cidx_v[pl.ds(4 * C + o, 16)] = r2 & 7
        if o < R3 + 16:
            r3 = sid * R3 + o + iota
            cidx_v[pl.ds(5 * C + o, 16)] = jnp.minimum(r3 >> 7, 12)
            cidx_v[pl.ds(6 * C + o, 16)] = (r3 >> 3) & 15
            cidx_v[pl.ds(7 * C + o, 16)] = (r3 >> 1) & 3
            cidx_v[pl.ds(8 * C + o, 16)] = r3 & 1

    g = pltpu.async_copy(w0.at[cidx_v.at[pl.ds(0 * C, R1)]],
                         acc0.at[pl.ds(0, R1)], sem_g0)
    g.wait()
    g1 = pltpu.async_copy(w2p.at[cidx_v.at[pl.ds(1 * C, R1)]],
                          acc0.at[pl.ds(0, R1)], sem_g0, add=True)
    g2 = pltpu.async_copy(w3.at[cidx_v.at[pl.ds(2 * C, R2)]], acc1, sem_g1)
    g1.wait()
    g2.wait()
    s1 = pltpu.async_copy(acc0.at[pl.ds(0, R1)],
                          spm1.at[pl.ds(sid * R1, R1)], sem_o0)
    g2b = pltpu.async_copy(w6p.at[cidx_v.at[pl.ds(3 * C, R2)]], acc1,
                           sem_g1, add=True)
    g2c = pltpu.async_copy(w4.at[cidx_v.at[pl.ds(4 * C, R2)]], acc1,
                           sem_g1, add=True)
    g2b.wait()
    g2c.wait()
    s2 = pltpu.async_copy(acc1, spm2.at[pl.ds(sid * R2, R2)], sem_o1)
    s1.wait()
    g3 = pltpu.async_copy(w7.at[cidx_v.at[pl.ds(5 * C, R3)]],
                          acc0.at[pl.ds(0, R3)], sem_g0)
    g3.wait()
    g3s = [pltpu.async_copy(w.at[cidx_v.at[pl.ds(s * C, R3)]],
                            acc0.at[pl.ds(0, R3)], sem_g0, add=True)
           for s, w in ((6, w8p), (7, w1), (8, w5))]
    for d in g3s:
        d.wait()
    s2.wait()
    pltpu.sync_copy(acc0.at[pl.ds(0, R3)], spm3.at[pl.ds(sid * R3, R3)])

    plsc.subcore_barrier()

    def cidx_body(j, carry):
        o = j * 16
        i0 = idx_v[pl.ds(0 * BPW_S + o, 16)]
        i2 = idx_v[pl.ds(2 * BPW_S + o, 16)]
        cidx_v[pl.ds(0 * BPW_S + o, 16)] = i0 * 16 + i2
        i3 = idx_v[pl.ds(3 * BPW_S + o, 16)]
        i6 = idx_v[pl.ds(6 * BPW_S + o, 16)]
        i4 = idx_v[pl.ds(4 * BPW_S + o, 16)]
        cidx_v[pl.ds(1 * BPW_S + o, 16)] = i3 * 128 + i6 * 8 + i4
        i7 = idx_v[pl.ds(7 * BPW_S + o, 16)]
        i8 = idx_v[pl.ds(8 * BPW_S + o, 16)]
        i1 = idx_v[pl.ds(1 * BPW_S + o, 16)]
        i5 = idx_v[pl.ds(5 * BPW_S + o, 16)]
        cidx_v[pl.ds(2 * BPW_S + o, 16)] = i7 * 128 + i8 * 8 + i1 * 2 + i5
        return carry

    lax.fori_loop(0, BPW_S // 16, cidx_body, 0)

    def fire_g(g, c, buf, add, n):
        cb = c * C
        return pltpu.async_copy(
            tables[g].at[cidx_v.at[pl.ds(g * BPW_S + cb, n)]],
            accs[buf].at[pl.ds(0, n)], gsems[buf], add=add)

    nchunks = CPW + 1
    sizes = [C] * CPW + [CTAIL]
    out_descs = [None, None]
    f0_descs = [None, None]
    f0_descs[0] = fire_g(0, 0, 0, False, sizes[0])
    for c in range(nchunks):
        buf = c % 2
        nbuf = (c + 1) % 2
        f0_descs[buf].wait()
        adds = [fire_g(g, c, buf, True, sizes[c])
                for g in range(1, NUM_GROUPS)]
        if c + 1 < nchunks:
            if out_descs[nbuf] is not None:
                out_descs[nbuf].wait()
            f0_descs[nbuf] = fire_g(0, c + 1, nbuf, False, sizes[c + 1])
        for d in adds:
            d.wait()
        cb = c * C
        out_descs[buf] = pltpu.async_copy(
            accs[buf].at[pl.ds(0, sizes[c])],
            out_hbm.at[pl.ds(base + cb, sizes[c])], osems[buf])
    for d in out_descs:
        if d is not None:
            d.wait()


_embed = functools.partial(
    pl.kernel,
    out_type=jax.ShapeDtypeStruct((N, D), jnp.float32),
    mesh=plsc.VectorSubcoreMesh(core_axis_name="c", subcore_axis_name="s"),
    compiler_params=pltpu.CompilerParams(use_tc_tiling_on_sc=False),
    scratch_types=[
        pltpu.VMEM((9 * BPW_S,), jnp.int32),
        pltpu.VMEM((NUM_GROUPS * BPW_S,), jnp.int32),
        pltpu.VMEM((C, D), jnp.float32),
        pltpu.VMEM((C, D), jnp.float32),
        pltpu.MemorySpace.VMEM_SHARED((T1_ROWS, D), jnp.float32),
        pltpu.MemorySpace.VMEM_SHARED((T2_ROWS, D), jnp.float32),
        pltpu.MemorySpace.VMEM_SHARED((T3_ROWS, D), jnp.float32),
        pltpu.SemaphoreType.DMA,
        pltpu.SemaphoreType.DMA,
        pltpu.SemaphoreType.DMA,
        pltpu.SemaphoreType.DMA,
    ],
)(_embed_body)


def kernel(idx_atomic_num, W_atomic_num, idx_chiral_tag, W_chiral_tag,
           idx_degree, W_degree, idx_formal_charge, W_formal_charge,
           idx_hybridization, W_hybridization, idx_is_aromatic, W_is_aromatic,
           idx_total_numHs, W_total_numHs, idx_explicit_valence,
           W_explicit_valence, idx_implicit_valence, W_implicit_valence):
    idxs = (idx_atomic_num, idx_chiral_tag, idx_degree, idx_formal_charge,
            idx_hybridization, idx_is_aromatic, idx_total_numHs,
            idx_explicit_valence, idx_implicit_valence)
    w0 = W_atomic_num[:119]
    w2p = jnp.pad(W_degree[:11], ((0, 5), (0, 0)))
    w3 = W_formal_charge[:16]
    w6p = jnp.pad(W_total_numHs[:9], ((0, 7), (0, 0)))
    w4 = W_hybridization[:8]
    w7 = W_explicit_valence[:13]
    w8p = jnp.pad(W_implicit_valence[:13], ((0, 3), (0, 0)))
    w1 = W_chiral_tag[:4]
    w5 = W_is_aromatic[:2]

    idx_stack = jnp.concatenate(
        [jnp.pad(i.astype(jnp.int32).reshape(NW, BPW),
                 ((0, 0), (0, BPW_S - BPW))).reshape(-1) for i in idxs])
    return _embed(idx_stack, w0, w2p, w3, w6p, w4, w7, w8p, w1, w5)


# speedup vs baseline: 1.7741x; 1.7741x over previous
import functools

import jax
import jax.numpy as jnp
from jax import lax
from jax.experimental import pallas as pl
from jax.experimental.pallas import tpu as pltpu
from jax.experimental.pallas import tpu_sc as plsc

N = 100000
D = 128
NC = 2
NS = 16
NW = NC * NS
C = 128
BPW = N // NW
CPW = BPW // C
CTAIL = BPW - CPW * C
BPW_S = 3136
NST = NW * BPW_S

T1_ROWS = 120 * 16
T2_ROWS = 16 * 16 * 8
T3_ROWS = 13 * 16 * 4 * 2
NUM_GROUPS = 3


def _build_body(w0_ref, w2p_ref, w3_ref, w6p_ref, w4_ref,
                w7_ref, w8p_ref, w1_ref, w5_ref, t1_ref, t2_ref, t3_ref):
    w2pv = w2p_ref[...]
    t1_ref[...] = jnp.concatenate(
        [w0_ref[min(a, 118):min(a, 118) + 1, :] + w2pv
         for a in range(120)], axis=0)
    w4v = w4_ref[...]
    inner128 = jnp.concatenate(
        [w6p_ref[j:j + 1, :] + w4v for j in range(16)], axis=0)
    t2_ref[...] = jnp.concatenate(
        [w3_ref[i:i + 1, :] + inner128 for i in range(16)], axis=0)
    inner8 = jnp.concatenate(
        [w1_ref[a:a + 1, :] + w5_ref[b:b + 1, :]
         for a in range(4) for b in range(2)], axis=0)
    mid = jnp.concatenate(
        [w8p_ref[j:j + 1, :] + inner8 for j in range(16)], axis=0)
    t3_ref[...] = jnp.concatenate(
        [w7_ref[i:i + 1, :] + mid for i in range(13)], axis=0)


def _build_tables(w0, w2p, w3, w6p, w4, w7, w8p, w1, w5):
    return pl.pallas_call(
        _build_body,
        out_shape=[
            jax.ShapeDtypeStruct((T1_ROWS, D), jnp.float32),
            jax.ShapeDtypeStruct((T2_ROWS, D), jnp.float32),
            jax.ShapeDtypeStruct((T3_ROWS, D), jnp.float32),
        ],
    )(w0, w2p, w3, w6p, w4, w7, w8p, w1, w5)


def _embed_body(idx_hbm, t1, t2, t3, out_hbm,
                idx_v, cidx_v, acc0, acc1, spm1, spm2, spm3,
                sem_g0, sem_g1, sem_o0, sem_o1):
    accs = (acc0, acc1)
    gsems = (sem_g0, sem_g1)
    osems = (sem_o0, sem_o1)
    sid = lax.axis_index("s")
    wid = sid * NC + lax.axis_index("c")
    base = wid * BPW

    tables = (spm1, spm2, spm3)
    for t_hbm, t_spm, rows in ((t1, spm1, T1_ROWS // NS),
                               (t2, spm2, T2_ROWS // NS),
                               (t3, spm3, T3_ROWS // NS)):
        pltpu.sync_copy(t_hbm.at[pl.ds(sid * rows, rows)],
                        t_spm.at[pl.ds(sid * rows, rows)])

    for f in range(9):
        pltpu.sync_copy(idx_hbm.at[pl.ds(f * NST + wid * BPW_S, BPW_S)],
                        idx_v.at[pl.ds(f * BPW_S, BPW_S)])

    plsc.subcore_barrier()

    def cidx_body(j, carry):
        o = j * 16
        i0 = idx_v[pl.ds(0 * BPW_S + o, 16)]
        i2 = idx_v[pl.ds(2 * BPW_S + o, 16)]
        cidx_v[pl.ds(0 * BPW_S + o, 16)] = i0 * 16 + i2
        i3 = idx_v[pl.ds(3 * BPW_S + o, 16)]
        i6 = idx_v[pl.ds(6 * BPW_S + o, 16)]
        i4 = idx_v[pl.ds(4 * BPW_S + o, 16)]
        cidx_v[pl.ds(1 * BPW_S + o, 16)] = i3 * 128 + i6 * 8 + i4
        i7 = idx_v[pl.ds(7 * BPW_S + o, 16)]
        i8 = idx_v[pl.ds(8 * BPW_S + o, 16)]
        i1 = idx_v[pl.ds(1 * BPW_S + o, 16)]
        i5 = idx_v[pl.ds(5 * BPW_S + o, 16)]
        cidx_v[pl.ds(2 * BPW_S + o, 16)] = i7 * 128 + i8 * 8 + i1 * 2 + i5
        return carry

    lax.fori_loop(0, BPW_S // 16, cidx_body, 0)

    tables_hbm = (t1, t2, t3)

    def fire_g(g, c, buf, add, n):
        cb = c * C
        src = tables_hbm if c % 4 == 3 else tables
        return pltpu.async_copy(
            src[g].at[cidx_v.at[pl.ds(g * BPW_S + cb, n)]],
            accs[buf].at[pl.ds(0, n)], gsems[buf], add=add)

    nchunks = CPW + 1
    sizes = [C] * CPW + [CTAIL]
    out_descs = [None, None]
    f0_descs = [None, None]
    f0_descs[0] = fire_g(0, 0, 0, False, sizes[0])
    for c in range(nchunks):
        buf = c % 2
        nbuf = (c + 1) % 2
        f0_descs[buf].wait()
        adds = [fire_g(g, c, buf, True, sizes[c])
                for g in range(1, NUM_GROUPS)]
        if c + 1 < nchunks:
            if out_descs[nbuf] is not None:
                out_descs[nbuf].wait()
            f0_descs[nbuf] = fire_g(0, c + 1, nbuf, False, sizes[c + 1])
        for d in adds:
            d.wait()
        cb = c * C
        out_descs[buf] = pltpu.async_copy(
            accs[buf].at[pl.ds(0, sizes[c])],
            out_hbm.at[pl.ds(base + cb, sizes[c])], osems[buf])
    for d in out_descs:
        if d is not None:
            d.wait()


_embed = functools.partial(
    pl.kernel,
    out_type=jax.ShapeDtypeStruct((N, D), jnp.float32),
    mesh=plsc.VectorSubcoreMesh(core_axis_name="c", subcore_axis_name="s"),
    compiler_params=pltpu.CompilerParams(use_tc_tiling_on_sc=False),
    scratch_types=[
        pltpu.VMEM((9 * BPW_S,), jnp.int32),
        pltpu.VMEM((NUM_GROUPS * BPW_S,), jnp.int32),
        pltpu.VMEM((C, D), jnp.float32),
        pltpu.VMEM((C, D), jnp.float32),
        pltpu.MemorySpace.VMEM_SHARED((T1_ROWS, D), jnp.float32),
        pltpu.MemorySpace.VMEM_SHARED((T2_ROWS, D), jnp.float32),
        pltpu.MemorySpace.VMEM_SHARED((T3_ROWS, D), jnp.float32),
        pltpu.SemaphoreType.DMA,
        pltpu.SemaphoreType.DMA,
        pltpu.SemaphoreType.DMA,
        pltpu.SemaphoreType.DMA,
    ],
)(_embed_body)


def kernel(idx_atomic_num, W_atomic_num, idx_chiral_tag, W_chiral_tag,
           idx_degree, W_degree, idx_formal_charge, W_formal_charge,
           idx_hybridization, W_hybridization, idx_is_aromatic, W_is_aromatic,
           idx_total_numHs, W_total_numHs, idx_explicit_valence,
           W_explicit_valence, idx_implicit_valence, W_implicit_valence):
    idxs = (idx_atomic_num, idx_chiral_tag, idx_degree, idx_formal_charge,
            idx_hybridization, idx_is_aromatic, idx_total_numHs,
            idx_explicit_valence, idx_implicit_valence)
    w0 = W_atomic_num[:119]
    w2p = jnp.pad(W_degree[:11], ((0, 5), (0, 0)))
    w3 = W_formal_charge[:16]
    w6p = jnp.pad(W_total_numHs[:9], ((0, 7), (0, 0)))
    w4 = W_hybridization[:8]
    w7 = W_explicit_valence[:13]
    w8p = jnp.pad(W_implicit_valence[:13], ((0, 3), (0, 0)))
    w1 = W_chiral_tag[:4]
    w5 = W_is_aromatic[:2]

    t1, t2, t3 = _build_tables(w0, w2p, w3, w6p, w4, w7, w8p, w1, w5)

    idx_stack = jnp.concatenate(
        [jnp.pad(i.astype(jnp.int32).reshape(NW, BPW),
                 ((0, 0), (0, BPW_S - BPW))).reshape(-1) for i in idxs])
    return _embed(idx_stack, t1, t2, t3)


# speedup vs baseline: 1.9302x; 1.0880x over previous
import functools

import jax
import jax.numpy as jnp
from jax import lax
from jax.experimental import pallas as pl
from jax.experimental.pallas import tpu as pltpu
from jax.experimental.pallas import tpu_sc as plsc

N = 100000
D = 128
NC = 2
NS = 16
NW = NC * NS
C = 128
BPW = N // NW
CPW = BPW // C
CTAIL = BPW - CPW * C
BPW_S = 3136
NST = NW * BPW_S

T1_ROWS = 120 * 16
T2_ROWS = 16 * 16 * 8
T3_ROWS = 1408
NUM_GROUPS = 3


def _build_body(w0_ref, w2p_ref, w3_ref, w6p_ref, w4_ref,
                w7_ref, w8p_ref, w1_ref, w5_ref, t1_ref, t2_ref, t3_ref):
    w2pv = w2p_ref[...]
    t1_ref[...] = jnp.concatenate(
        [w0_ref[min(a, 118):min(a, 118) + 1, :] + w2pv
         for a in range(120)], axis=0)
    w4v = w4_ref[...]
    inner128 = jnp.concatenate(
        [w6p_ref[j:j + 1, :] + w4v for j in range(16)], axis=0)
    t2_ref[...] = jnp.concatenate(
        [w3_ref[i:i + 1, :] + inner128 for i in range(16)], axis=0)
    inner8 = jnp.concatenate(
        [w1_ref[a:a + 1, :] + w5_ref[b:b + 1, :]
         for a in range(4) for b in range(2)], axis=0)
    mid = jnp.concatenate(
        [w8p_ref[j:j + 1, :] + inner8 for j in range(13)], axis=0)
    t3_ref[...] = jnp.concatenate(
        [w7_ref[i:i + 1, :] + mid for i in range(13)]
        + [jnp.zeros((T3_ROWS - 13 * 104, D), jnp.float32)], axis=0)


def _build_tables(w0, w2p, w3, w6p, w4, w7, w8p, w1, w5):
    return pl.pallas_call(
        _build_body,
        out_shape=[
            jax.ShapeDtypeStruct((T1_ROWS, D), jnp.float32),
            jax.ShapeDtypeStruct((T2_ROWS, D), jnp.float32),
            jax.ShapeDtypeStruct((T3_ROWS, D), jnp.float32),
        ],
    )(w0, w2p, w3, w6p, w4, w7, w8p, w1, w5)


def _embed_body(idx_hbm, t1, t2, t3, out_hbm,
                idx_v, cidx_v, acc0, acc1, acc2, spm1, spm2, spm3,
                sem_g0, sem_g1, sem_g2, sem_o0, sem_o1, sem_o2):
    accs = (acc0, acc1, acc2)
    gsems = (sem_g0, sem_g1, sem_g2)
    osems = (sem_o0, sem_o1, sem_o2)
    sid = lax.axis_index("s")
    wid = sid * NC + lax.axis_index("c")
    base = wid * BPW

    tables = (spm1, spm2, spm3)
    for t_hbm, t_spm, rows in ((t1, spm1, T1_ROWS // NS),
                               (t2, spm2, T2_ROWS // NS),
                               (t3, spm3, T3_ROWS // NS)):
        pltpu.sync_copy(t_hbm.at[pl.ds(sid * rows, rows)],
                        t_spm.at[pl.ds(sid * rows, rows)])

    for f in range(9):
        pltpu.sync_copy(idx_hbm.at[pl.ds(f * NST + wid * BPW_S, BPW_S)],
                        idx_v.at[pl.ds(f * BPW_S, BPW_S)])

    plsc.subcore_barrier()

    def cidx_body(j, carry):
        o = j * 16
        i0 = idx_v[pl.ds(0 * BPW_S + o, 16)]
        i2 = idx_v[pl.ds(2 * BPW_S + o, 16)]
        cidx_v[pl.ds(0 * BPW_S + o, 16)] = i0 * 16 + i2
        i3 = idx_v[pl.ds(3 * BPW_S + o, 16)]
        i6 = idx_v[pl.ds(6 * BPW_S + o, 16)]
        i4 = idx_v[pl.ds(4 * BPW_S + o, 16)]
        cidx_v[pl.ds(1 * BPW_S + o, 16)] = i3 * 128 + i6 * 8 + i4
        i7 = idx_v[pl.ds(7 * BPW_S + o, 16)]
        i8 = idx_v[pl.ds(8 * BPW_S + o, 16)]
        i1 = idx_v[pl.ds(1 * BPW_S + o, 16)]
        i5 = idx_v[pl.ds(5 * BPW_S + o, 16)]
        cidx_v[pl.ds(2 * BPW_S + o, 16)] = i7 * 104 + i8 * 8 + i1 * 2 + i5
        return carry

    lax.fori_loop(0, BPW_S // 16, cidx_body, 0)

    tables_hbm = (t1, t2, t3)

    def fire_g(g, c, buf, add, n):
        cb = c * C
        src = tables_hbm if c % 4 == 3 else tables
        return pltpu.async_copy(
            src[g].at[cidx_v.at[pl.ds(g * BPW_S + cb, n)]],
            accs[buf].at[pl.ds(0, n)], gsems[buf], add=add)

    nchunks = CPW + 1
    sizes = [C] * CPW + [CTAIL]
    NB = 3
    out_descs = [None] * NB
    f0_descs = [None] * NB
    adds_descs = [None] * nchunks

    def fire_out(c):
        cb = c * C
        return pltpu.async_copy(
            accs[c % NB].at[pl.ds(0, sizes[c])],
            out_hbm.at[pl.ds(base + cb, sizes[c])], osems[c % NB])

    f0_descs[0] = fire_g(0, 0, 0 % NB, False, sizes[0])
    for c in range(nchunks):
        buf = c % NB
        f0_descs[buf].wait()
        adds_descs[c] = [fire_g(g, c, buf, True, sizes[c])
                         for g in range(1, NUM_GROUPS)]
        if c + 1 < nchunks:
            b1 = (c + 1) % NB
            if out_descs[b1] is not None:
                out_descs[b1].wait()
            f0_descs[b1] = fire_g(0, c + 1, b1, False, sizes[c + 1])
        if c >= 1:
            for d in adds_descs[c - 1]:
                d.wait()
            out_descs[(c - 1) % NB] = fire_out(c - 1)
    for d in adds_descs[nchunks - 1]:
        d.wait()
    out_descs[(nchunks - 1) % NB] = fire_out(nchunks - 1)
    for d in out_descs:
        if d is not None:
            d.wait()


_embed = functools.partial(
    pl.kernel,
    out_type=jax.ShapeDtypeStruct((N, D), jnp.float32),
    mesh=plsc.VectorSubcoreMesh(core_axis_name="c", subcore_axis_name="s"),
    compiler_params=pltpu.CompilerParams(use_tc_tiling_on_sc=False),
    scratch_types=[
        pltpu.VMEM((9 * BPW_S,), jnp.int32),
        pltpu.VMEM((NUM_GROUPS * BPW_S,), jnp.int32),
        pltpu.VMEM((C, D), jnp.float32),
        pltpu.VMEM((C, D), jnp.float32),
        pltpu.VMEM((C, D), jnp.float32),
        pltpu.MemorySpace.VMEM_SHARED((T1_ROWS, D), jnp.float32),
        pltpu.MemorySpace.VMEM_SHARED((T2_ROWS, D), jnp.float32),
        pltpu.MemorySpace.VMEM_SHARED((T3_ROWS, D), jnp.float32),
        pltpu.SemaphoreType.DMA,
        pltpu.SemaphoreType.DMA,
        pltpu.SemaphoreType.DMA,
        pltpu.SemaphoreType.DMA,
        pltpu.SemaphoreType.DMA,
        pltpu.SemaphoreType.DMA,
    ],
)(_embed_body)


def kernel(idx_atomic_num, W_atomic_num, idx_chiral_tag, W_chiral_tag,
           idx_degree, W_degree, idx_formal_charge, W_formal_charge,
           idx_hybridization, W_hybridization, idx_is_aromatic, W_is_aromatic,
           idx_total_numHs, W_total_numHs, idx_explicit_valence,
           W_explicit_valence, idx_implicit_valence, W_implicit_valence):
    idxs = (idx_atomic_num, idx_chiral_tag, idx_degree, idx_formal_charge,
            idx_hybridization, idx_is_aromatic, idx_total_numHs,
            idx_explicit_valence, idx_implicit_valence)
    w0 = W_atomic_num[:119]
    w2p = jnp.pad(W_degree[:11], ((0, 5), (0, 0)))
    w3 = W_formal_charge[:16]
    w6p = jnp.pad(W_total_numHs[:9], ((0, 7), (0, 0)))
    w4 = W_hybridization[:8]
    w7 = W_explicit_valence[:13]
    w8p = jnp.pad(W_implicit_valence[:13], ((0, 3), (0, 0)))
    w1 = W_chiral_tag[:4]
    w5 = W_is_aromatic[:2]

    t1, t2, t3 = _build_tables(w0, w2p, w3, w6p, w4, w7, w8p, w1, w5)

    idx_stack = jnp.concatenate(
        [jnp.pad(i.astype(jnp.int32).reshape(NW, BPW),
                 ((0, 0), (0, BPW_S - BPW))).reshape(-1) for i in idxs])
    return _embed(idx_stack, t1, t2, t3)


# speedup vs baseline: 1.9832x; 1.0275x over previous
import functools

import jax
import jax.numpy as jnp
from jax import lax
from jax.experimental import pallas as pl
from jax.experimental.pallas import tpu as pltpu
from jax.experimental.pallas import tpu_sc as plsc

N = 100000
D = 128
NC = 2
NS = 16
NW = NC * NS
C = 128
BPW = N // NW
CPW = BPW // C
CTAIL = BPW - CPW * C
BPW_S = 3136
NST = NW * BPW_S

T1_ROWS = 120 * 16
T2_ROWS = 16 * 16 * 8
T3_ROWS = 1408
NUM_GROUPS = 3


def _build_body(w0_ref, w2p_ref, w3_ref, w6p_ref, w4_ref,
                w7_ref, w8p_ref, w1_ref, w5_ref, t1_ref, t2_ref, t3_ref):
    w2pv = w2p_ref[...]
    t1_ref[...] = jnp.concatenate(
        [w0_ref[min(a, 118):min(a, 118) + 1, :] + w2pv
         for a in range(120)], axis=0)
    w4v = w4_ref[...]
    inner128 = jnp.concatenate(
        [w6p_ref[j:j + 1, :] + w4v for j in range(16)], axis=0)
    t2_ref[...] = jnp.concatenate(
        [w3_ref[i:i + 1, :] + inner128 for i in range(16)], axis=0)
    inner8 = jnp.concatenate(
        [w1_ref[a:a + 1, :] + w5_ref[b:b + 1, :]
         for a in range(4) for b in range(2)], axis=0)
    mid = jnp.concatenate(
        [w8p_ref[j:j + 1, :] + inner8 for j in range(13)], axis=0)
    t3_ref[...] = jnp.concatenate(
        [w7_ref[i:i + 1, :] + mid for i in range(13)]
        + [jnp.zeros((T3_ROWS - 13 * 104, D), jnp.float32)], axis=0)


def _build_tables(w0, w2p, w3, w6p, w4, w7, w8p, w1, w5):
    return pl.pallas_call(
        _build_body,
        out_shape=[
            jax.ShapeDtypeStruct((T1_ROWS, D), jnp.float32),
            jax.ShapeDtypeStruct((T2_ROWS, D), jnp.float32),
            jax.ShapeDtypeStruct((T3_ROWS, D), jnp.float32),
        ],
    )(w0, w2p, w3, w6p, w4, w7, w8p, w1, w5)


def _embed_body(idx_hbm, t1, t2, t3, out_hbm,
                idx_v, cidx_v, acc0, acc1, acc2, spm1, spm2, spm3,
                sem_g0, sem_g1, sem_g2, sem_o0, sem_o1, sem_o2):
    accs = (acc0, acc1, acc2)
    gsems = (sem_g0, sem_g1, sem_g2)
    osems = (sem_o0, sem_o1, sem_o2)
    sid = lax.axis_index("s")
    wid = sid * NC + lax.axis_index("c")
    base = wid * BPW

    tables = (spm1, spm2, spm3)
    for t_hbm, t_spm, rows in ((t1, spm1, T1_ROWS // NS),
                               (t2, spm2, T2_ROWS // NS),
                               (t3, spm3, T3_ROWS // NS)):
        pltpu.sync_copy(t_hbm.at[pl.ds(sid * rows, rows)],
                        t_spm.at[pl.ds(sid * rows, rows)])

    for f in range(9):
        pltpu.sync_copy(idx_hbm.at[pl.ds(f * NST + wid * BPW_S, BPW_S)],
                        idx_v.at[pl.ds(f * BPW_S, BPW_S)])

    plsc.subcore_barrier()

    def cidx_body(j, carry):
        o = j * 16
        i0 = idx_v[pl.ds(0 * BPW_S + o, 16)]
        i2 = idx_v[pl.ds(2 * BPW_S + o, 16)]
        cidx_v[pl.ds(0 * BPW_S + o, 16)] = i0 * 16 + i2
        i3 = idx_v[pl.ds(3 * BPW_S + o, 16)]
        i6 = idx_v[pl.ds(6 * BPW_S + o, 16)]
        i4 = idx_v[pl.ds(4 * BPW_S + o, 16)]
        cidx_v[pl.ds(1 * BPW_S + o, 16)] = i3 * 128 + i6 * 8 + i4
        i7 = idx_v[pl.ds(7 * BPW_S + o, 16)]
        i8 = idx_v[pl.ds(8 * BPW_S + o, 16)]
        i1 = idx_v[pl.ds(1 * BPW_S + o, 16)]
        i5 = idx_v[pl.ds(5 * BPW_S + o, 16)]
        cidx_v[pl.ds(2 * BPW_S + o, 16)] = i7 * 104 + i8 * 8 + i1 * 2 + i5
        return carry

    lax.fori_loop(0, BPW_S // 16, cidx_body, 0)

    tables_hbm = (t1, t2, t3)

    def fire_g(g, c, buf, add, n):
        cb = c * C
        src = tables_hbm if c % 3 == 2 else tables
        return pltpu.async_copy(
            src[g].at[cidx_v.at[pl.ds(g * BPW_S + cb, n)]],
            accs[buf].at[pl.ds(0, n)], gsems[buf], add=add)

    nchunks = CPW + 1
    sizes = [C] * CPW + [CTAIL]
    NB = 3
    out_descs = [None] * NB
    f0_descs = [None] * NB
    adds_descs = [None] * nchunks

    def fire_out(c):
        cb = c * C
        return pltpu.async_copy(
            accs[c % NB].at[pl.ds(0, sizes[c])],
            out_hbm.at[pl.ds(base + cb, sizes[c])], osems[c % NB])

    f0_descs[0] = fire_g(0, 0, 0 % NB, False, sizes[0])
    for c in range(nchunks):
        buf = c % NB
        f0_descs[buf].wait()
        adds_descs[c] = [fire_g(g, c, buf, True, sizes[c])
                         for g in range(1, NUM_GROUPS)]
        if c + 1 < nchunks:
            b1 = (c + 1) % NB
            if out_descs[b1] is not None:
                out_descs[b1].wait()
            f0_descs[b1] = fire_g(0, c + 1, b1, False, sizes[c + 1])
        if c >= 1:
            for d in adds_descs[c - 1]:
                d.wait()
            out_descs[(c - 1) % NB] = fire_out(c - 1)
    for d in adds_descs[nchunks - 1]:
        d.wait()
    out_descs[(nchunks - 1) % NB] = fire_out(nchunks - 1)
    for d in out_descs:
        if d is not None:
            d.wait()


_embed = functools.partial(
    pl.kernel,
    out_type=jax.ShapeDtypeStruct((N, D), jnp.float32),
    mesh=plsc.VectorSubcoreMesh(core_axis_name="c", subcore_axis_name="s"),
    compiler_params=pltpu.CompilerParams(use_tc_tiling_on_sc=False),
    scratch_types=[
        pltpu.VMEM((9 * BPW_S,), jnp.int32),
        pltpu.VMEM((NUM_GROUPS * BPW_S,), jnp.int32),
        pltpu.VMEM((C, D), jnp.float32),
        pltpu.VMEM((C, D), jnp.float32),
        pltpu.VMEM((C, D), jnp.float32),
        pltpu.MemorySpace.VMEM_SHARED((T1_ROWS, D), jnp.float32),
        pltpu.MemorySpace.VMEM_SHARED((T2_ROWS, D), jnp.float32),
        pltpu.MemorySpace.VMEM_SHARED((T3_ROWS, D), jnp.float32),
        pltpu.SemaphoreType.DMA,
        pltpu.SemaphoreType.DMA,
        pltpu.SemaphoreType.DMA,
        pltpu.SemaphoreType.DMA,
        pltpu.SemaphoreType.DMA,
        pltpu.SemaphoreType.DMA,
    ],
)(_embed_body)


def kernel(idx_atomic_num, W_atomic_num, idx_chiral_tag, W_chiral_tag,
           idx_degree, W_degree, idx_formal_charge, W_formal_charge,
           idx_hybridization, W_hybridization, idx_is_aromatic, W_is_aromatic,
           idx_total_numHs, W_total_numHs, idx_explicit_valence,
           W_explicit_valence, idx_implicit_valence, W_implicit_valence):
    idxs = (idx_atomic_num, idx_chiral_tag, idx_degree, idx_formal_charge,
            idx_hybridization, idx_is_aromatic, idx_total_numHs,
            idx_explicit_valence, idx_implicit_valence)
    w0 = W_atomic_num[:119]
    w2p = jnp.pad(W_degree[:11], ((0, 5), (0, 0)))
    w3 = W_formal_charge[:16]
    w6p = jnp.pad(W_total_numHs[:9], ((0, 7), (0, 0)))
    w4 = W_hybridization[:8]
    w7 = W_explicit_valence[:13]
    w8p = jnp.pad(W_implicit_valence[:13], ((0, 3), (0, 0)))
    w1 = W_chiral_tag[:4]
    w5 = W_is_aromatic[:2]

    t1, t2, t3 = _build_tables(w0, w2p, w3, w6p, w4, w7, w8p, w1, w5)

    idx_stack = jnp.concatenate(
        [jnp.pad(i.astype(jnp.int32).reshape(NW, BPW),
                 ((0, 0), (0, BPW_S - BPW))).reshape(-1) for i in idxs])
    return _embed(idx_stack, t1, t2, t3)
